# baseline (speedup 1.0000x reference)
import functools

import jax
import jax.numpy as jnp
from jax import lax
from jax.experimental import pallas as pl
from jax.experimental.pallas import tpu as pltpu
from jax.experimental.pallas import tpu_sc as plsc

N = 10000
NP = 10240
NB = 1024
NBLK = NP // NB
FEAT = 256
H = 512
ND = 128
G = 128
E = 160000
EP = 163840
ER = EP // 128
TILES = 32
EPT_ROWS = ER // TILES
STRIPE = NP // 16


@functools.lru_cache(maxsize=None)
def _sc_agg(C):
    mesh = plsc.VectorSubcoreMesh(core_axis_name="c", subcore_axis_name="s")

    @functools.partial(
        pl.kernel,
        out_type=jax.ShapeDtypeStruct((2 * C * NP, 128), jnp.float32),
        mesh=mesh,
        scratch_types=[
            pltpu.VMEM((EPT_ROWS, 128), jnp.int32),
            pltpu.VMEM((EPT_ROWS, 128), jnp.int32),
            pltpu.VMEM((128, 128), jnp.float32),
            pltpu.VMEM((128, 128), jnp.float32),
            pltpu.VMEM_SHARED((NP, 128), jnp.float32),
            pltpu.SemaphoreType.DMA,
            pltpu.SemaphoreType.DMA,
        ],
    )
    def agg(h_hbm, src_hbm, dst_hbm, zeros_hbm, out_hbm,
            src_v, dst_v, buf0, buf1, acc, sem0, sem1):
        core = lax.axis_index("c")
        sub = lax.axis_index("s")
        tid = core * 16 + sub
        erow0 = tid * EPT_ROWS
        stripe0 = sub * STRIPE
        pltpu.sync_copy(dst_hbm.at[pl.ds(erow0, EPT_ROWS)], dst_v)
        pltpu.sync_copy(src_hbm.at[pl.ds(erow0, EPT_ROWS)], src_v)
        pltpu.sync_copy(zeros_hbm.at[pl.ds(stripe0, STRIPE)],
                        acc.at[pl.ds(stripe0, STRIPE)])
        plsc.subcore_barrier()
        pltpu.async_copy(h_hbm.at[0].at[src_v.at[0]], buf0, sem0)
        pltpu.async_copy(h_hbm.at[0].at[src_v.at[1]], buf1, sem1)
        for ci in range(C):
            hc = h_hbm.at[ci]

            def pair(p, carry):
                j0 = 2 * p
                j1 = j0 + 1
                pltpu.make_async_copy(hc.at[src_v.at[j0]], buf0, sem0).wait()
                pltpu.sync_copy(buf0, acc.at[dst_v.at[j0]], add=True)

                @pl.when(j0 + 2 < EPT_ROWS)
                def _():
                    pltpu.async_copy(hc.at[src_v.at[j0 + 2]], buf0, sem0)

                pltpu.make_async_copy(hc.at[src_v.at[j1]], buf1, sem1).wait()
                pltpu.sync_copy(buf1, acc.at[dst_v.at[j1]], add=True)

                @pl.when(j1 + 2 < EPT_ROWS)
                def _():
                    pltpu.async_copy(hc.at[src_v.at[j1 + 2]], buf1, sem1)

                return carry

            lax.fori_loop(0, EPT_ROWS // 2, pair, 0)
            plsc.subcore_barrier()
            if ci + 1 < C:
                hn = h_hbm.at[ci + 1]
                pltpu.async_copy(hn.at[src_v.at[0]], buf0, sem0)
                pltpu.async_copy(hn.at[src_v.at[1]], buf1, sem1)
            off = core * (C * NP) + ci * NP + stripe0
            pltpu.sync_copy(acc.at[pl.ds(stripe0, STRIPE)],
                            out_hbm.at[pl.ds(off, STRIPE)])
            plsc.subcore_barrier()

    return agg


def _row_mask(i):
    rows = i * NB + lax.broadcasted_iota(jnp.int32, (NB, 1), 0)
    return rows < N


def _mlp_stats_body(h_ref, agg_ref, w1_ref, b1_ref, w2_ref, b2_ref,
                    h2_ref, st_ref, *, Cin):
    i = pl.program_id(0)
    pref = agg_ref[0] + agg_ref[1]
    parts = []
    for c in range(Cin):
        d = pref[c] if c == 0 else pref[c] - pref[c - 1]
        parts.append(h_ref[c] + d)
    hcat = jnp.concatenate(parts, axis=1)
    z = jnp.dot(hcat.astype(jnp.bfloat16), w1_ref[...],
                preferred_element_type=jnp.float32)
    z = jnp.maximum(z + b1_ref[...], 0.0)
    h2 = jnp.dot(z.astype(jnp.bfloat16), w2_ref[...],
                 preferred_element_type=jnp.float32)
    h2 = jnp.maximum(h2 + b2_ref[...], 0.0)
    h2_ref[...] = h2.astype(jnp.bfloat16)
    h2m = jnp.where(_row_mask(i), h2, 0.0)
    s1 = jnp.sum(h2m, axis=0, keepdims=True)
    s2 = jnp.sum(h2m * h2m, axis=0, keepdims=True)
    st = jnp.concatenate([s1, s2, jnp.zeros((6, H), jnp.float32)], axis=0)

    @pl.when(i == 0)
    def _():
        st_ref[...] = jnp.zeros_like(st_ref)

    st_ref[...] += st


def _mlp_stats(h_ch, agg, w1, b1, w2, b2):
    Cin = h_ch.shape[0]
    return pl.pallas_call(
        functools.partial(_mlp_stats_body, Cin=Cin),
        grid=(NBLK,),
        in_specs=[
            pl.BlockSpec((Cin, NB, 128), lambda i: (0, i, 0)),
            pl.BlockSpec((2, Cin, NB, 128), lambda i: (0, 0, i, 0)),
            pl.BlockSpec((Cin * 128, H), lambda i: (0, 0)),
            pl.BlockSpec((1, H), lambda i: (0, 0)),
            pl.BlockSpec((H, H), lambda i: (0, 0)),
            pl.BlockSpec((1, H), lambda i: (0, 0)),
        ],
        out_specs=[
            pl.BlockSpec((NB, H), lambda i: (i, 0)),
            pl.BlockSpec((8, H), lambda i: (0, 0)),
        ],
        out_shape=[
            jax.ShapeDtypeStruct((NP, H), jnp.bfloat16),
            jax.ShapeDtypeStruct((8, H), jnp.float32),
        ],
    )(h_ch, agg, w1, b1, w2, b2)


def _norm_body(h2_ref, st_ref, g_ref, b_ref, out_ref):
    i = pl.program_id(0)
    st = st_ref[...]
    mean = st[0:1, :] * (1.0 / N)
    var = st[1:2, :] * (1.0 / N) - mean * mean
    scale = g_ref[...] * lax.rsqrt(var + 1e-5)
    shift = b_ref[...] - mean * scale
    hn = h2_ref[...].astype(jnp.float32) * scale + shift
    hn = jnp.where(_row_mask(i), hn, 0.0)
    for c in range(H // 128):
        out_ref[c] = hn[:, c * 128:(c + 1) * 128]


def _norm_chunk(h2, stats, gamma, beta):
    return pl.pallas_call(
        _norm_body,
        grid=(NBLK,),
        in_specs=[
            pl.BlockSpec((NB, H), lambda i: (i, 0)),
            pl.BlockSpec((8, H), lambda i: (0, 0)),
            pl.BlockSpec((1, H), lambda i: (0, 0)),
            pl.BlockSpec((1, H), lambda i: (0, 0)),
        ],
        out_specs=pl.BlockSpec((H // 128, NB, 128), lambda i: (0, i, 0)),
        out_shape=jax.ShapeDtypeStruct((H // 128, NP, 128), jnp.float32),
    )(h2, stats, gamma, beta)


def _hb_partial_body(f_ref, wh_ref, *rest, has_prev):
    out_ref = rest[-1]
    cat = jnp.concatenate([f_ref[c] for c in range(4)], axis=1)
    r = jnp.dot(cat.astype(jnp.bfloat16), wh_ref[...],
                preferred_element_type=jnp.float32)
    if has_prev:
        r = r + rest[0][...]
    out_ref[...] = r


def _hb_partial(feat, whl, prev=None):
    has_prev = prev is not None
    in_specs = [
        pl.BlockSpec((4, NB, 128), lambda i: (0, i, 0)),
        pl.BlockSpec((H, H), lambda i: (0, 0)),
    ]
    args = [feat, whl]
    if has_prev:
        in_specs.append(pl.BlockSpec((NB, H), lambda i: (i, 0)))
        args.append(prev)
    return pl.pallas_call(
        functools.partial(_hb_partial_body, has_prev=has_prev),
        grid=(NBLK,),
        in_specs=in_specs,
        out_specs=pl.BlockSpec((NB, H), lambda i: (i, 0)),
        out_shape=jax.ShapeDtypeStruct((NP, H), jnp.float32),
    )(*args)


def _heads_body(f3_ref, hb12_ref, wh_ref, bh_ref, eps_ref, batch_ref,
                zn_ref, nkl_ref, si_ref, sm_ref):
    i = pl.program_id(0)
    hcat = jnp.concatenate([f3_ref[c] for c in range(4)], axis=1)
    hb = jnp.dot(hcat.astype(jnp.bfloat16), wh_ref[...],
                 preferred_element_type=jnp.float32)
    hb = hb + hb12_ref[...] + bh_ref[...]
    nmu = hb[:, 0:128]
    nlv = hb[:, 128:256]
    cmu = hb[:, 256:384]
    clv = hb[:, 384:512]
    zn_ref[...] = nmu + eps_ref[...] * jnp.exp(0.5 * nlv)
    mask = _row_mask(i)
    kl = jnp.where(mask, 1.0 + nlv - nmu * nmu - jnp.exp(nlv), 0.0)
    lane = lax.broadcasted_iota(jnp.int32, (NB, G), 1)
    oh = jnp.where(jnp.logical_and(mask, batch_ref[...] == lane), 1.0, 0.0)
    iv = jnp.exp(-clv)
    dn = (((0,), (0,)), ((), ()))
    si = lax.dot_general(oh, iv, dn, preferred_element_type=jnp.float32)
    sm = lax.dot_general(oh, cmu * iv, dn, preferred_element_type=jnp.float32)

    @pl.when(i == 0)
    def _():
        nkl_ref[...] = jnp.zeros_like(nkl_ref)
        si_ref[...] = jnp.zeros_like(si_ref)
        sm_ref[...] = jnp.zeros_like(sm_ref)

    nkl_ref[...] += (-0.5 * jnp.sum(kl)).reshape(1, 1)
    si_ref[...] += si
    sm_ref[...] += sm


def _heads(f3, hb12, wh3, bh, eps_n, batch_b):
    return pl.pallas_call(
        _heads_body,
        grid=(NBLK,),
        in_specs=[
            pl.BlockSpec((4, NB, 128), lambda i: (0, i, 0)),
            pl.BlockSpec((NB, H), lambda i: (i, 0)),
            pl.BlockSpec((H, H), lambda i: (0, 0)),
            pl.BlockSpec((1, H), lambda i: (0, 0)),
            pl.BlockSpec((NB, 128), lambda i: (i, 0)),
            pl.BlockSpec((NB, G), lambda i: (i, 0)),
        ],
        out_specs=[
            pl.BlockSpec((NB, 128), lambda i: (i, 0)),
            pl.BlockSpec((1, 1), lambda i: (0, 0)),
            pl.BlockSpec((G, 128), lambda i: (0, 0)),
            pl.BlockSpec((G, 128), lambda i: (0, 0)),
        ],
        out_shape=[
            jax.ShapeDtypeStruct((NP, 128), jnp.float32),
            jax.ShapeDtypeStruct((1, 1), jnp.float32),
            jax.ShapeDtypeStruct((G, 128), jnp.float32),
            jax.ShapeDtypeStruct((G, 128), jnp.float32),
        ],
    )(f3, hb12, wh3, bh, eps_n, batch_b)


def _decoder_body(zn_ref, batch_ref, x_ref, si_ref, sm_ref, eg_ref, nkl_ref,
                  w1_ref, b1_ref, w2_ref, b2_ref, loss_ref, zct_ref, ckl_ref):
    i = pl.program_id(0)

    @pl.when(i == 0)
    def _():
        g_var = 1.0 / si_ref[...]
        g_mu = g_var * sm_ref[...]
        g_lv = jnp.log(g_var)
        colmask = lax.broadcasted_iota(jnp.int32, (G, 128), 1) < 64
        terms = jnp.where(colmask, 1.0 + g_lv - g_mu * g_mu - g_var, 0.0)
        ckl_ref[0, 0] = -0.5 * jnp.sum(terms)
        zct_ref[...] = g_mu + eg_ref[...] * jnp.exp(0.5 * g_lv)
        loss_ref[...] = jnp.zeros_like(loss_ref)

    mask = _row_mask(i)
    lane = lax.broadcasted_iota(jnp.int32, (NB, G), 1)
    oh = jnp.where(jnp.logical_and(mask, batch_ref[...] == lane), 1.0, 0.0)
    zc = jnp.dot(oh, zct_ref[...], preferred_element_type=jnp.float32)
    cat = jnp.concatenate([zn_ref[...], zc], axis=1)
    a = jnp.dot(cat.astype(jnp.bfloat16), w1_ref[...],
                preferred_element_type=jnp.float32)
    a = jnp.maximum(a + b1_ref[...], 0.0)
    recon = jnp.dot(a.astype(jnp.bfloat16), w2_ref[...],
                    preferred_element_type=jnp.float32)
    recon = recon + b2_ref[...]
    diff = jnp.where(mask, recon - x_ref[...], 0.0)
    loss_ref[...] += jnp.sum(diff * diff).reshape(1, 1)

    @pl.when(i == NBLK - 1)
    def _():
        loss_ref[...] = (loss_ref[...] * (1.0 / (N * FEAT))
                         + ckl_ref[0, 0] + nkl_ref[...])


def _decoder(zn, batch_b, x_pad, si, sm, eps_g, nkl, w1, b1, w2, b2):
    return pl.pallas_call(
        _decoder_body,
        grid=(NBLK,),
        in_specs=[
            pl.BlockSpec((NB, 128), lambda i: (i, 0)),
            pl.BlockSpec((NB, G), lambda i: (i, 0)),
            pl.BlockSpec((NB, FEAT), lambda i: (i, 0)),
            pl.BlockSpec((G, 128), lambda i: (0, 0)),
            pl.BlockSpec((G, 128), lambda i: (0, 0)),
            pl.BlockSpec((G, 128), lambda i: (0, 0)),
            pl.BlockSpec((1, 1), lambda i: (0, 0)),
            pl.BlockSpec((FEAT, H), lambda i: (0, 0)),
            pl.BlockSpec((1, H), lambda i: (0, 0)),
            pl.BlockSpec((H, FEAT), lambda i: (0, 0)),
            pl.BlockSpec((1, FEAT), lambda i: (0, 0)),
        ],
        out_specs=pl.BlockSpec((1, 1), lambda i: (0, 0)),
        out_shape=jax.ShapeDtypeStruct((1, 1), jnp.float32),
        scratch_shapes=[
            pltpu.VMEM((G, 128), jnp.float32),
            pltpu.SMEM((1, 1), jnp.float32),
        ],
    )(zn, batch_b, x_pad, si, sm, eps_g, nkl, w1, b1, w2, b2)


def kernel(x, params, edge_index, batch, num_graphs):
    f32 = jnp.float32
    src = edge_index[0].astype(jnp.int32)
    dst = edge_index[1].astype(jnp.int32)
    pad = N + (jnp.arange(EP - E, dtype=jnp.int32) % (NP - N))
    src_rows = jnp.concatenate([src, pad]).reshape(ER, 128)
    dst_rows = jnp.concatenate([dst, pad]).reshape(ER, 128)
    zeros_np = jnp.zeros((NP, 128), f32)

    x_pad = jnp.pad(x, ((0, NP - N), (0, 0)))
    x_ch = x_pad.reshape(NP, 2, 128).transpose(1, 0, 2)

    batch_b = jnp.broadcast_to(
        jnp.pad(batch.astype(jnp.int32), (0, NP - N))[:, None], (NP, G))
    eps_n = jax.random.normal(jax.random.key(1), (N, ND), dtype=f32)
    eps_n = jnp.pad(eps_n, ((0, NP - N), (0, 0)))
    eps_g = jax.random.normal(jax.random.key(2), (G, 64), dtype=f32)
    eps_g = jnp.pad(eps_g, ((0, 0), (0, 64)))

    z64 = jnp.zeros((3 * H, 64), f32)
    wh = jnp.concatenate([
        params["node_mu"]["W"], params["node_logvar"]["W"],
        params["class_mu"]["W"], z64, params["class_logvar"]["W"], z64,
    ], axis=1).astype(jnp.bfloat16)
    bz = jnp.zeros((64,), f32)
    bh = jnp.concatenate([
        params["node_mu"]["b"], params["node_logvar"]["b"],
        params["class_mu"]["b"], bz, params["class_logvar"]["b"], bz,
    ]).reshape(1, H)

    w1d = jnp.concatenate([params["dec1"]["W"], jnp.zeros((64, H), f32)],
                          axis=0).astype(jnp.bfloat16)
    b1d = params["dec1"]["b"].reshape(1, H)
    w2d = params["dec2"]["W"].astype(jnp.bfloat16)
    b2d = params["dec2"]["b"].reshape(1, FEAT)

    h_ch = x_ch
    hb12 = None
    for li, layer in enumerate(params["gin"]):
        Cin = h_ch.shape[0]
        agg = _sc_agg(Cin)(h_ch, src_rows, dst_rows, zeros_np)
        agg = agg.reshape(2, Cin, NP, 128)
        h2, stats = _mlp_stats(h_ch, agg,
                               layer["lin1"]["W"].astype(jnp.bfloat16),
                               layer["lin1"]["b"].reshape(1, H),
                               layer["lin2"]["W"].astype(jnp.bfloat16),
                               layer["lin2"]["b"].reshape(1, H))
        h_ch = _norm_chunk(h2, stats, layer["gamma"].reshape(1, H),
                           layer["beta"].reshape(1, H))
        if li < 2:
            hb12 = _hb_partial(h_ch, wh[li * H:(li + 1) * H], hb12)

    zn, nkl, si, sm = _heads(h_ch, hb12, wh[2 * H:], bh, eps_n, batch_b)
    loss = _decoder(zn, batch_b, x_pad, si, sm, eps_g, nkl,
                    w1d, b1d, w2d, b2d)
    return loss[0, 0]


# speedup vs baseline: 1.0425x; 1.0425x over previous
import functools

import jax
import jax.numpy as jnp
from jax import lax
from jax.experimental import pallas as pl
from jax.experimental.pallas import tpu as pltpu
from jax.experimental.pallas import tpu_sc as plsc

N = 10000
NP = 10240
NB = 1024
NBLK = NP // NB
FEAT = 256
H = 512
ND = 128
G = 128
E = 160000
EP = 163840
ER = EP // 128
TILES = 32
EPT_ROWS = ER // TILES
DROWS = ER // 16
STRIPE = NP // 16


@functools.lru_cache(maxsize=None)
def _sc_agg(C):
    CP = C // 2
    mesh = plsc.VectorSubcoreMesh(core_axis_name="c", subcore_axis_name="s")

    @functools.partial(
        pl.kernel,
        out_type=jax.ShapeDtypeStruct((C * NP, 128), jnp.float32),
        mesh=mesh,
        scratch_types=[
            pltpu.VMEM((EPT_ROWS, 128), jnp.int32),
            pltpu.VMEM((DROWS, 128), jnp.int32),
            pltpu.VMEM((128, 128), jnp.float32),
            pltpu.VMEM((128, 128), jnp.float32),
            pltpu.VMEM_SHARED((NP, 128), jnp.float32),
            pltpu.SemaphoreType.DMA,
            pltpu.SemaphoreType.DMA,
        ],
    )
    def agg(h_hbm, src_hbm, dst_hbm, zeros_hbm, out_hbm,
            src_v, dst_v, buf0, buf1, acc, sem0, sem1):
        core = lax.axis_index("c")
        sub = lax.axis_index("s")
        drow0 = sub * DROWS
        stripe0 = sub * STRIPE
        pltpu.sync_copy(dst_hbm.at[pl.ds(drow0, DROWS)], dst_v)
        pltpu.sync_copy(zeros_hbm.at[pl.ds(stripe0, STRIPE)],
                        acc.at[pl.ds(stripe0, STRIPE)])
        plsc.subcore_barrier()
        for ck in range(CP):
            cidx = core * CP + ck
            for half in range(2):
                srow0 = cidx * ER + drow0 + half * EPT_ROWS
                pltpu.sync_copy(src_hbm.at[pl.ds(srow0, EPT_ROWS)], src_v)
                pltpu.async_copy(h_hbm.at[src_v.at[0]], buf0, sem0)
                pltpu.async_copy(h_hbm.at[src_v.at[1]], buf1, sem1)
                dbase = half * EPT_ROWS

                def pair(p, carry):
                    j0 = 2 * p
                    j1 = j0 + 1
                    pltpu.make_async_copy(h_hbm.at[src_v.at[j0]], buf0,
                                          sem0).wait()
                    pltpu.sync_copy(buf0, acc.at[dst_v.at[dbase + j0]],
                                    add=True)

                    @pl.when(j0 + 2 < EPT_ROWS)
                    def _():
                        pltpu.async_copy(h_hbm.at[src_v.at[j0 + 2]], buf0,
                                         sem0)

                    pltpu.make_async_copy(h_hbm.at[src_v.at[j1]], buf1,
                                          sem1).wait()
                    pltpu.sync_copy(buf1, acc.at[dst_v.at[dbase + j1]],
                                    add=True)

                    @pl.when(j1 + 2 < EPT_ROWS)
                    def _():
                        pltpu.async_copy(h_hbm.at[src_v.at[j1 + 2]], buf1,
                                         sem1)

                    return carry

                lax.fori_loop(0, EPT_ROWS // 2, pair, 0)
            plsc.subcore_barrier()
            off = cidx * NP + stripe0
            pltpu.sync_copy(acc.at[pl.ds(stripe0, STRIPE)],
                            out_hbm.at[pl.ds(off, STRIPE)])
            plsc.subcore_barrier()

    return agg


def _row_mask(i):
    rows = i * NB + lax.broadcasted_iota(jnp.int32, (NB, 1), 0)
    return rows < N


def _mlp_stats_body(h_ref, agg_ref, w1_ref, b1_ref, w2_ref, b2_ref,
                    h2_ref, st_ref, *, Cin):
    i = pl.program_id(0)
    pref = agg_ref[...]
    CP = max(Cin // 2, 1)
    parts = []
    for c in range(Cin):
        d = pref[c] if c % CP == 0 else pref[c] - pref[c - 1]
        parts.append(h_ref[c] + d)
    hcat = jnp.concatenate(parts, axis=1)
    z = jnp.dot(hcat.astype(jnp.bfloat16), w1_ref[...],
                preferred_element_type=jnp.float32)
    z = jnp.maximum(z + b1_ref[...], 0.0)
    h2 = jnp.dot(z.astype(jnp.bfloat16), w2_ref[...],
                 preferred_element_type=jnp.float32)
    h2 = jnp.maximum(h2 + b2_ref[...], 0.0)
    h2_ref[...] = h2.astype(jnp.bfloat16)
    h2m = jnp.where(_row_mask(i), h2, 0.0)
    s1 = jnp.sum(h2m, axis=0, keepdims=True)
    s2 = jnp.sum(h2m * h2m, axis=0, keepdims=True)
    st = jnp.concatenate([s1, s2, jnp.zeros((6, H), jnp.float32)], axis=0)

    @pl.when(i == 0)
    def _():
        st_ref[...] = jnp.zeros_like(st_ref)

    st_ref[...] += st


def _mlp_stats(h_ch, agg, w1, b1, w2, b2):
    Cin = h_ch.shape[0]
    return pl.pallas_call(
        functools.partial(_mlp_stats_body, Cin=Cin),
        grid=(NBLK,),
        in_specs=[
            pl.BlockSpec((Cin, NB, 128), lambda i: (0, i, 0)),
            pl.BlockSpec((Cin, NB, 128), lambda i: (0, i, 0)),
            pl.BlockSpec((Cin * 128, H), lambda i: (0, 0)),
            pl.BlockSpec((1, H), lambda i: (0, 0)),
            pl.BlockSpec((H, H), lambda i: (0, 0)),
            pl.BlockSpec((1, H), lambda i: (0, 0)),
        ],
        out_specs=[
            pl.BlockSpec((NB, H), lambda i: (i, 0)),
            pl.BlockSpec((8, H), lambda i: (0, 0)),
        ],
        out_shape=[
            jax.ShapeDtypeStruct((NP, H), jnp.bfloat16),
            jax.ShapeDtypeStruct((8, H), jnp.float32),
        ],
    )(h_ch, agg, w1, b1, w2, b2)


def _norm_body(h2_ref, st_ref, g_ref, b_ref, out_ref):
    i = pl.program_id(0)
    st = st_ref[...]
    mean = st[0:1, :] * (1.0 / N)
    var = st[1:2, :] * (1.0 / N) - mean * mean
    scale = g_ref[...] * lax.rsqrt(var + 1e-5)
    shift = b_ref[...] - mean * scale
    hn = h2_ref[...].astype(jnp.float32) * scale + shift
    hn = jnp.where(_row_mask(i), hn, 0.0)
    for c in range(H // 128):
        out_ref[c] = hn[:, c * 128:(c + 1) * 128]


def _norm_chunk(h2, stats, gamma, beta):
    return pl.pallas_call(
        _norm_body,
        grid=(NBLK,),
        in_specs=[
            pl.BlockSpec((NB, H), lambda i: (i, 0)),
            pl.BlockSpec((8, H), lambda i: (0, 0)),
            pl.BlockSpec((1, H), lambda i: (0, 0)),
            pl.BlockSpec((1, H), lambda i: (0, 0)),
        ],
        out_specs=pl.BlockSpec((H // 128, NB, 128), lambda i: (0, i, 0)),
        out_shape=jax.ShapeDtypeStruct((H // 128, NP, 128), jnp.float32),
    )(h2, stats, gamma, beta)


def _heads_body(f1_ref, f2_ref, f3_ref, wh_ref, bh_ref, eps_ref, batch_ref,
                zn_ref, nkl_ref, si_ref, sm_ref):
    i = pl.program_id(0)
    parts = [f1_ref[c] for c in range(4)] + [f2_ref[c] for c in range(4)] \
        + [f3_ref[c] for c in range(4)]
    hcat = jnp.concatenate(parts, axis=1)
    hb = jnp.dot(hcat.astype(jnp.bfloat16), wh_ref[...],
                 preferred_element_type=jnp.float32)
    hb = hb + bh_ref[...]
    nmu = hb[:, 0:128]
    nlv = hb[:, 128:256]
    cmu = hb[:, 256:384]
    clv = hb[:, 384:512]
    zn_ref[...] = nmu + eps_ref[...] * jnp.exp(0.5 * nlv)
    mask = _row_mask(i)
    kl = jnp.where(mask, 1.0 + nlv - nmu * nmu - jnp.exp(nlv), 0.0)
    lane = lax.broadcasted_iota(jnp.int32, (NB, G), 1)
    oh = jnp.where(jnp.logical_and(mask, batch_ref[...] == lane), 1.0, 0.0)
    iv = jnp.exp(-clv)
    dn = (((0,), (0,)), ((), ()))
    si = lax.dot_general(oh, iv, dn, preferred_element_type=jnp.float32)
    sm = lax.dot_general(oh, cmu * iv, dn, preferred_element_type=jnp.float32)

    @pl.when(i == 0)
    def _():
        nkl_ref[...] = jnp.zeros_like(nkl_ref)
        si_ref[...] = jnp.zeros_like(si_ref)
        sm_ref[...] = jnp.zeros_like(sm_ref)

    nkl_ref[...] += (-0.5 * jnp.sum(kl)).reshape(1, 1)
    si_ref[...] += si
    sm_ref[...] += sm


def _heads(f1, f2, f3, wh, bh, eps_n, batch_b):
    return pl.pallas_call(
        _heads_body,
        grid=(NBLK,),
        in_specs=[
            pl.BlockSpec((4, NB, 128), lambda i: (0, i, 0)),
            pl.BlockSpec((4, NB, 128), lambda i: (0, i, 0)),
            pl.BlockSpec((4, NB, 128), lambda i: (0, i, 0)),
            pl.BlockSpec((3 * H, H), lambda i: (0, 0)),
            pl.BlockSpec((1, H), lambda i: (0, 0)),
            pl.BlockSpec((NB, 128), lambda i: (i, 0)),
            pl.BlockSpec((NB, G), lambda i: (i, 0)),
        ],
        out_specs=[
            pl.BlockSpec((NB, 128), lambda i: (i, 0)),
            pl.BlockSpec((1, 1), lambda i: (0, 0)),
            pl.BlockSpec((G, 128), lambda i: (0, 0)),
            pl.BlockSpec((G, 128), lambda i: (0, 0)),
        ],
        out_shape=[
            jax.ShapeDtypeStruct((NP, 128), jnp.float32),
            jax.ShapeDtypeStruct((1, 1), jnp.float32),
            jax.ShapeDtypeStruct((G, 128), jnp.float32),
            jax.ShapeDtypeStruct((G, 128), jnp.float32),
        ],
    )(f1, f2, f3, wh, bh, eps_n, batch_b)


def _decoder_body(zn_ref, batch_ref, x_ref, si_ref, sm_ref, eg_ref, nkl_ref,
                  w1_ref, b1_ref, w2_ref, b2_ref, loss_ref, zct_ref, ckl_ref):
    i = pl.program_id(0)

    @pl.when(i == 0)
    def _():
        g_var = 1.0 / si_ref[...]
        g_mu = g_var * sm_ref[...]
        g_lv = jnp.log(g_var)
        colmask = lax.broadcasted_iota(jnp.int32, (G, 128), 1) < 64
        terms = jnp.where(colmask, 1.0 + g_lv - g_mu * g_mu - g_var, 0.0)
        ckl_ref[0, 0] = -0.5 * jnp.sum(terms)
        zct_ref[...] = g_mu + eg_ref[...] * jnp.exp(0.5 * g_lv)
        loss_ref[...] = jnp.zeros_like(loss_ref)

    mask = _row_mask(i)
    lane = lax.broadcasted_iota(jnp.int32, (NB, G), 1)
    oh = jnp.where(jnp.logical_and(mask, batch_ref[...] == lane), 1.0, 0.0)
    zc = jnp.dot(oh, zct_ref[...], preferred_element_type=jnp.float32)
    cat = jnp.concatenate([zn_ref[...], zc], axis=1)
    a = jnp.dot(cat.astype(jnp.bfloat16), w1_ref[...],
                preferred_element_type=jnp.float32)
    a = jnp.maximum(a + b1_ref[...], 0.0)
    recon = jnp.dot(a.astype(jnp.bfloat16), w2_ref[...],
                    preferred_element_type=jnp.float32)
    recon = recon + b2_ref[...]
    diff = jnp.where(mask, recon - x_ref[...], 0.0)
    loss_ref[...] += jnp.sum(diff * diff).reshape(1, 1)

    @pl.when(i == NBLK - 1)
    def _():
        loss_ref[...] = (loss_ref[...] * (1.0 / (N * FEAT))
                         + ckl_ref[0, 0] + nkl_ref[...])


def _decoder(zn, batch_b, x_pad, si, sm, eps_g, nkl, w1, b1, w2, b2):
    return pl.pallas_call(
        _decoder_body,
        grid=(NBLK,),
        in_specs=[
            pl.BlockSpec((NB, 128), lambda i: (i, 0)),
            pl.BlockSpec((NB, G), lambda i: (i, 0)),
            pl.BlockSpec((NB, FEAT), lambda i: (i, 0)),
            pl.BlockSpec((G, 128), lambda i: (0, 0)),
            pl.BlockSpec((G, 128), lambda i: (0, 0)),
            pl.BlockSpec((G, 128), lambda i: (0, 0)),
            pl.BlockSpec((1, 1), lambda i: (0, 0)),
            pl.BlockSpec((FEAT, H), lambda i: (0, 0)),
            pl.BlockSpec((1, H), lambda i: (0, 0)),
            pl.BlockSpec((H, FEAT), lambda i: (0, 0)),
            pl.BlockSpec((1, FEAT), lambda i: (0, 0)),
        ],
        out_specs=pl.BlockSpec((1, 1), lambda i: (0, 0)),
        out_shape=jax.ShapeDtypeStruct((1, 1), jnp.float32),
        scratch_shapes=[
            pltpu.VMEM((G, 128), jnp.float32),
            pltpu.SMEM((1, 1), jnp.float32),
        ],
    )(zn, batch_b, x_pad, si, sm, eps_g, nkl, w1, b1, w2, b2)


def kernel(x, params, edge_index, batch, num_graphs):
    f32 = jnp.float32
    src = edge_index[0].astype(jnp.int32)
    dst = edge_index[1].astype(jnp.int32)
    pad = N + (jnp.arange(EP - E, dtype=jnp.int32) % (NP - N))
    srcp = jnp.concatenate([src, pad])
    offs = (jnp.arange(4, dtype=jnp.int32) * NP)[:, None]
    src_rows4 = (srcp[None, :] + offs).reshape(4 * ER, 128)
    dst_rows = jnp.concatenate([dst, pad]).reshape(ER, 128)
    zeros_np = jnp.zeros((NP, 128), f32)

    x_pad = jnp.pad(x, ((0, NP - N), (0, 0)))
    x_ch = x_pad.reshape(NP, 2, 128).transpose(1, 0, 2)

    batch_b = jnp.broadcast_to(
        jnp.pad(batch.astype(jnp.int32), (0, NP - N))[:, None], (NP, G))
    eps_n = jax.random.normal(jax.random.key(1), (N, ND), dtype=f32)
    eps_n = jnp.pad(eps_n, ((0, NP - N), (0, 0)))
    eps_g = jax.random.normal(jax.random.key(2), (G, 64), dtype=f32)
    eps_g = jnp.pad(eps_g, ((0, 0), (0, 64)))

    z64 = jnp.zeros((3 * H, 64), f32)
    wh = jnp.concatenate([
        params["node_mu"]["W"], params["node_logvar"]["W"],
        params["class_mu"]["W"], z64, params["class_logvar"]["W"], z64,
    ], axis=1).astype(jnp.bfloat16)
    bz = jnp.zeros((64,), f32)
    bh = jnp.concatenate([
        params["node_mu"]["b"], params["node_logvar"]["b"],
        params["class_mu"]["b"], bz, params["class_logvar"]["b"], bz,
    ]).reshape(1, H)

    w1d = jnp.concatenate([params["dec1"]["W"], jnp.zeros((64, H), f32)],
                          axis=0).astype(jnp.bfloat16)
    b1d = params["dec1"]["b"].reshape(1, H)
    w2d = params["dec2"]["W"].astype(jnp.bfloat16)
    b2d = params["dec2"]["b"].reshape(1, FEAT)

    h_ch = x_ch
    feats = []
    for layer in params["gin"]:
        Cin = h_ch.shape[0]
        agg = _sc_agg(Cin)(h_ch.reshape(Cin * NP, 128),
                           src_rows4[:Cin * ER], dst_rows, zeros_np)
        agg = agg.reshape(Cin, NP, 128)
        h2, stats = _mlp_stats(h_ch, agg,
                               layer["lin1"]["W"].astype(jnp.bfloat16),
                               layer["lin1"]["b"].reshape(1, H),
                               layer["lin2"]["W"].astype(jnp.bfloat16),
                               layer["lin2"]["b"].reshape(1, H))
        h_ch = _norm_chunk(h2, stats, layer["gamma"].reshape(1, H),
                           layer["beta"].reshape(1, H))
        feats.append(h_ch)

    zn, nkl, si, sm = _heads(feats[0], feats[1], feats[2], wh, bh,
                             eps_n, batch_b)
    loss = _decoder(zn, batch_b, x_pad, si, sm, eps_g, nkl,
                    w1d, b1d, w2d, b2d)
    return loss[0, 0]


# speedup vs baseline: 1.0655x; 1.0221x over previous
import functools

import jax
import jax.numpy as jnp
from jax import lax
from jax.experimental import pallas as pl
from jax.experimental.pallas import tpu as pltpu
from jax.experimental.pallas import tpu_sc as plsc

N = 10000
NP = 10240
NB = 1024
NBLK = NP // NB
FEAT = 256
H = 512
ND = 128
G = 128
E = 160000
EP = 163840
ER = EP // 128
TILES = 32
EPT_ROWS = ER // TILES
DROWS = ER // 16
STRIPE = NP // 16


@functools.lru_cache(maxsize=None)
def _sc_agg(C):
    CP = C // 2
    mesh = plsc.VectorSubcoreMesh(core_axis_name="c", subcore_axis_name="s")

    @functools.partial(
        pl.kernel,
        out_type=jax.ShapeDtypeStruct((C * NP, 128), jnp.float32),
        mesh=mesh,
        scratch_types=[
            pltpu.VMEM((EPT_ROWS, 128), jnp.int32),
            pltpu.VMEM((DROWS, 128), jnp.int32),
            pltpu.VMEM((128, 128), jnp.float32),
            pltpu.VMEM((128, 128), jnp.float32),
            pltpu.VMEM_SHARED((NP, 128), jnp.float32),
            pltpu.SemaphoreType.DMA,
            pltpu.SemaphoreType.DMA,
        ],
    )
    def agg(h_hbm, src_hbm, dst_hbm, zeros_hbm, out_hbm,
            src_v, dst_v, buf0, buf1, acc, sem0, sem1):
        core = lax.axis_index("c")
        sub = lax.axis_index("s")
        drow0 = sub * DROWS
        stripe0 = sub * STRIPE
        pltpu.sync_copy(dst_hbm.at[pl.ds(drow0, DROWS)], dst_v)
        pltpu.sync_copy(zeros_hbm.at[pl.ds(stripe0, STRIPE)],
                        acc.at[pl.ds(stripe0, STRIPE)])
        plsc.subcore_barrier()
        for ck in range(CP):
            cidx = core * CP + ck
            for half in range(2):
                srow0 = cidx * ER + drow0 + half * EPT_ROWS
                pltpu.sync_copy(src_hbm.at[pl.ds(srow0, EPT_ROWS)], src_v)
                pltpu.async_copy(h_hbm.at[src_v.at[0]], buf0, sem0)
                pltpu.async_copy(h_hbm.at[src_v.at[1]], buf1, sem1)
                dbase = half * EPT_ROWS

                def pair(p, carry):
                    j0 = 2 * p
                    j1 = j0 + 1
                    pltpu.make_async_copy(h_hbm.at[src_v.at[j0]], buf0,
                                          sem0).wait()
                    pltpu.sync_copy(buf0, acc.at[dst_v.at[dbase + j0]],
                                    add=True)

                    @pl.when(j0 + 2 < EPT_ROWS)
                    def _():
                        pltpu.async_copy(h_hbm.at[src_v.at[j0 + 2]], buf0,
                                         sem0)

                    pltpu.make_async_copy(h_hbm.at[src_v.at[j1]], buf1,
                                          sem1).wait()
                    pltpu.sync_copy(buf1, acc.at[dst_v.at[dbase + j1]],
                                    add=True)

                    @pl.when(j1 + 2 < EPT_ROWS)
                    def _():
                        pltpu.async_copy(h_hbm.at[src_v.at[j1 + 2]], buf1,
                                         sem1)

                    return carry

                lax.fori_loop(0, EPT_ROWS // 2, pair, 0)
            plsc.subcore_barrier()
            off = cidx * NP + stripe0
            pltpu.sync_copy(acc.at[pl.ds(stripe0, STRIPE)],
                            out_hbm.at[pl.ds(off, STRIPE)])
            plsc.subcore_barrier()

    return agg


def _row_mask(i):
    rows = i * NB + lax.broadcasted_iota(jnp.int32, (NB, 1), 0)
    return rows < N


def _mlp_stats_body(h_ref, agg_ref, w1_ref, b1_ref, w2_ref, b2_ref,
                    h2_ref, st_ref, *, Cin):
    i = pl.program_id(0)
    pref = agg_ref[...]
    CP = max(Cin // 2, 1)
    parts = []
    for c in range(Cin):
        d = pref[c] if c % CP == 0 else pref[c] - pref[c - 1]
        parts.append(h_ref[c] + d)
    hcat = jnp.concatenate(parts, axis=1)
    z = jnp.dot(hcat.astype(jnp.bfloat16), w1_ref[...],
                preferred_element_type=jnp.float32)
    z = jnp.maximum(z + b1_ref[...], 0.0)
    h2 = jnp.dot(z.astype(jnp.bfloat16), w2_ref[...],
                 preferred_element_type=jnp.float32)
    h2 = jnp.maximum(h2 + b2_ref[...], 0.0)
    h2_ref[...] = h2.astype(jnp.bfloat16)
    h2m = jnp.where(_row_mask(i), h2, 0.0)
    s1 = jnp.sum(h2m, axis=0, keepdims=True)
    s2 = jnp.sum(h2m * h2m, axis=0, keepdims=True)
    st = jnp.concatenate([s1, s2, jnp.zeros((6, H), jnp.float32)], axis=0)

    @pl.when(i == 0)
    def _():
        st_ref[...] = jnp.zeros_like(st_ref)

    st_ref[...] += st


def _mlp_stats(h_ch, agg, w1, b1, w2, b2):
    Cin = h_ch.shape[0]
    return pl.pallas_call(
        functools.partial(_mlp_stats_body, Cin=Cin),
        grid=(NBLK,),
        in_specs=[
            pl.BlockSpec((Cin, NB, 128), lambda i: (0, i, 0)),
            pl.BlockSpec((Cin, NB, 128), lambda i: (0, i, 0)),
            pl.BlockSpec((Cin * 128, H), lambda i: (0, 0)),
            pl.BlockSpec((1, H), lambda i: (0, 0)),
            pl.BlockSpec((H, H), lambda i: (0, 0)),
            pl.BlockSpec((1, H), lambda i: (0, 0)),
        ],
        out_specs=[
            pl.BlockSpec((NB, H), lambda i: (i, 0)),
            pl.BlockSpec((8, H), lambda i: (0, 0)),
        ],
        out_shape=[
            jax.ShapeDtypeStruct((NP, H), jnp.bfloat16),
            jax.ShapeDtypeStruct((8, H), jnp.float32),
        ],
    )(h_ch, agg, w1, b1, w2, b2)


def _norm_body(h2_ref, st_ref, g_ref, b_ref, out_ref):
    i = pl.program_id(0)
    st = st_ref[...]
    mean = st[0:1, :] * (1.0 / N)
    var = st[1:2, :] * (1.0 / N) - mean * mean
    scale = g_ref[...] * lax.rsqrt(var + 1e-5)
    shift = b_ref[...] - mean * scale
    hn = h2_ref[...].astype(jnp.float32) * scale + shift
    hn = jnp.where(_row_mask(i), hn, 0.0)
    for c in range(H // 128):
        out_ref[c] = hn[:, c * 128:(c + 1) * 128]


def _norm_chunk(h2, stats, gamma, beta):
    return pl.pallas_call(
        _norm_body,
        grid=(NBLK,),
        in_specs=[
            pl.BlockSpec((NB, H), lambda i: (i, 0)),
            pl.BlockSpec((8, H), lambda i: (0, 0)),
            pl.BlockSpec((1, H), lambda i: (0, 0)),
            pl.BlockSpec((1, H), lambda i: (0, 0)),
        ],
        out_specs=pl.BlockSpec((H // 128, NB, 128), lambda i: (0, i, 0)),
        out_shape=jax.ShapeDtypeStruct((H // 128, NP, 128), jnp.float32),
    )(h2, stats, gamma, beta)


def _heads_body(f1_ref, f2_ref, h23_ref, st3_ref, g3_ref, b3_ref,
                wh_ref, bh_ref, eps_ref, batch_ref,
                zn_ref, nkl_ref, si_ref, sm_ref):
    i = pl.program_id(0)
    st = st3_ref[...]
    mean3 = st[0:1, :] * (1.0 / N)
    var3 = st[1:2, :] * (1.0 / N) - mean3 * mean3
    scale3 = g3_ref[...] * lax.rsqrt(var3 + 1e-5)
    shift3 = b3_ref[...] - mean3 * scale3
    hn3 = h23_ref[...].astype(jnp.float32) * scale3 + shift3
    hn3 = jnp.where(_row_mask(i), hn3, 0.0)
    parts = [f1_ref[c] for c in range(4)] + [f2_ref[c] for c in range(4)] \
        + [hn3[:, k * 128:(k + 1) * 128] for k in range(4)]
    hcat = jnp.concatenate(parts, axis=1)
    hb = jnp.dot(hcat.astype(jnp.bfloat16), wh_ref[...],
                 preferred_element_type=jnp.float32)
    hb = hb + bh_ref[...]
    nmu = hb[:, 0:128]
    nlv = hb[:, 128:256]
    cmu = hb[:, 256:384]
    clv = hb[:, 384:512]
    zn_ref[...] = nmu + eps_ref[...] * jnp.exp(0.5 * nlv)
    mask = _row_mask(i)
    kl = jnp.where(mask, 1.0 + nlv - nmu * nmu - jnp.exp(nlv), 0.0)
    lane = lax.broadcasted_iota(jnp.int32, (NB, G), 1)
    oh = jnp.where(jnp.logical_and(mask, batch_ref[...] == lane), 1.0, 0.0)
    iv = jnp.exp(-clv)
    dn = (((0,), (0,)), ((), ()))
    si = lax.dot_general(oh, iv, dn, preferred_element_type=jnp.float32)
    sm = lax.dot_general(oh, cmu * iv, dn, preferred_element_type=jnp.float32)

    @pl.when(i == 0)
    def _():
        nkl_ref[...] = jnp.zeros_like(nkl_ref)
        si_ref[...] = jnp.zeros_like(si_ref)
        sm_ref[...] = jnp.zeros_like(sm_ref)

    nkl_ref[...] += (-0.5 * jnp.sum(kl)).reshape(1, 1)
    si_ref[...] += si
    sm_ref[...] += sm


def _heads(f1, f2, h2_3, st3, g3, b3, wh, bh, eps_n, batch_b):
    return pl.pallas_call(
        _heads_body,
        grid=(NBLK,),
        in_specs=[
            pl.BlockSpec((4, NB, 128), lambda i: (0, i, 0)),
            pl.BlockSpec((4, NB, 128), lambda i: (0, i, 0)),
            pl.BlockSpec((NB, H), lambda i: (i, 0)),
            pl.BlockSpec((8, H), lambda i: (0, 0)),
            pl.BlockSpec((1, H), lambda i: (0, 0)),
            pl.BlockSpec((1, H), lambda i: (0, 0)),
            pl.BlockSpec((3 * H, H), lambda i: (0, 0)),
            pl.BlockSpec((1, H), lambda i: (0, 0)),
            pl.BlockSpec((NB, 128), lambda i: (i, 0)),
            pl.BlockSpec((NB, G), lambda i: (i, 0)),
        ],
        out_specs=[
            pl.BlockSpec((NB, 128), lambda i: (i, 0)),
            pl.BlockSpec((1, 1), lambda i: (0, 0)),
            pl.BlockSpec((G, 128), lambda i: (0, 0)),
            pl.BlockSpec((G, 128), lambda i: (0, 0)),
        ],
        out_shape=[
            jax.ShapeDtypeStruct((NP, 128), jnp.float32),
            jax.ShapeDtypeStruct((1, 1), jnp.float32),
            jax.ShapeDtypeStruct((G, 128), jnp.float32),
            jax.ShapeDtypeStruct((G, 128), jnp.float32),
        ],
    )(f1, f2, h2_3, st3, g3, b3, wh, bh, eps_n, batch_b)


def _decoder_body(zn_ref, batch_ref, x_ref, si_ref, sm_ref, eg_ref, nkl_ref,
                  w1_ref, b1_ref, w2_ref, b2_ref, loss_ref, zct_ref, ckl_ref):
    i = pl.program_id(0)

    @pl.when(i == 0)
    def _():
        g_var = 1.0 / si_ref[...]
        g_mu = g_var * sm_ref[...]
        g_lv = jnp.log(g_var)
        colmask = lax.broadcasted_iota(jnp.int32, (G, 128), 1) < 64
        terms = jnp.where(colmask, 1.0 + g_lv - g_mu * g_mu - g_var, 0.0)
        ckl_ref[0, 0] = -0.5 * jnp.sum(terms)
        zct_ref[...] = g_mu + eg_ref[...] * jnp.exp(0.5 * g_lv)
        loss_ref[...] = jnp.zeros_like(loss_ref)

    mask = _row_mask(i)
    lane = lax.broadcasted_iota(jnp.int32, (NB, G), 1)
    oh = jnp.where(jnp.logical_and(mask, batch_ref[...] == lane), 1.0, 0.0)
    zc = jnp.dot(oh, zct_ref[...], preferred_element_type=jnp.float32)
    cat = jnp.concatenate([zn_ref[...], zc], axis=1)
    a = jnp.dot(cat.astype(jnp.bfloat16), w1_ref[...],
                preferred_element_type=jnp.float32)
    a = jnp.maximum(a + b1_ref[...], 0.0)
    recon = jnp.dot(a.astype(jnp.bfloat16), w2_ref[...],
                    preferred_element_type=jnp.float32)
    recon = recon + b2_ref[...]
    diff = jnp.where(mask, recon - x_ref[...], 0.0)
    loss_ref[...] += jnp.sum(diff * diff).reshape(1, 1)

    @pl.when(i == NBLK - 1)
    def _():
        loss_ref[...] = (loss_ref[...] * (1.0 / (N * FEAT))
                         + ckl_ref[0, 0] + nkl_ref[...])


def _decoder(zn, batch_b, x_pad, si, sm, eps_g, nkl, w1, b1, w2, b2):
    return pl.pallas_call(
        _decoder_body,
        grid=(NBLK,),
        in_specs=[
            pl.BlockSpec((NB, 128), lambda i: (i, 0)),
            pl.BlockSpec((NB, G), lambda i: (i, 0)),
            pl.BlockSpec((NB, FEAT), lambda i: (i, 0)),
            pl.BlockSpec((G, 128), lambda i: (0, 0)),
            pl.BlockSpec((G, 128), lambda i: (0, 0)),
            pl.BlockSpec((G, 128), lambda i: (0, 0)),
            pl.BlockSpec((1, 1), lambda i: (0, 0)),
            pl.BlockSpec((FEAT, H), lambda i: (0, 0)),
            pl.BlockSpec((1, H), lambda i: (0, 0)),
            pl.BlockSpec((H, FEAT), lambda i: (0, 0)),
            pl.BlockSpec((1, FEAT), lambda i: (0, 0)),
        ],
        out_specs=pl.BlockSpec((1, 1), lambda i: (0, 0)),
        out_shape=jax.ShapeDtypeStruct((1, 1), jnp.float32),
        scratch_shapes=[
            pltpu.VMEM((G, 128), jnp.float32),
            pltpu.SMEM((1, 1), jnp.float32),
        ],
    )(zn, batch_b, x_pad, si, sm, eps_g, nkl, w1, b1, w2, b2)


def kernel(x, params, edge_index, batch, num_graphs):
    f32 = jnp.float32
    src = edge_index[0].astype(jnp.int32)
    dst = edge_index[1].astype(jnp.int32)
    pad = N + (jnp.arange(EP - E, dtype=jnp.int32) % (NP - N))
    srcp = jnp.concatenate([src, pad])
    offs = (jnp.arange(4, dtype=jnp.int32) * NP)[:, None]
    src_rows4 = (srcp[None, :] + offs).reshape(4 * ER, 128)
    dst_rows = jnp.concatenate([dst, pad]).reshape(ER, 128)
    zeros_np = jnp.zeros((NP, 128), f32)

    x_pad = jnp.pad(x, ((0, NP - N), (0, 0)))
    x_ch = x_pad.reshape(NP, 2, 128).transpose(1, 0, 2)

    batch_b = jnp.broadcast_to(
        jnp.pad(batch.astype(jnp.int32), (0, NP - N))[:, None], (NP, G))
    eps_n = jax.random.normal(jax.random.key(1), (N, ND), dtype=f32)
    eps_n = jnp.pad(eps_n, ((0, NP - N), (0, 0)))
    eps_g = jax.random.normal(jax.random.key(2), (G, 64), dtype=f32)
    eps_g = jnp.pad(eps_g, ((0, 0), (0, 64)))

    z64 = jnp.zeros((3 * H, 64), f32)
    wh = jnp.concatenate([
        params["node_mu"]["W"], params["node_logvar"]["W"],
        params["class_mu"]["W"], z64, params["class_logvar"]["W"], z64,
    ], axis=1).astype(jnp.bfloat16)
    bz = jnp.zeros((64,), f32)
    bh = jnp.concatenate([
        params["node_mu"]["b"], params["node_logvar"]["b"],
        params["class_mu"]["b"], bz, params["class_logvar"]["b"], bz,
    ]).reshape(1, H)

    w1d = jnp.concatenate([params["dec1"]["W"], jnp.zeros((64, H), f32)],
                          axis=0).astype(jnp.bfloat16)
    b1d = params["dec1"]["b"].reshape(1, H)
    w2d = params["dec2"]["W"].astype(jnp.bfloat16)
    b2d = params["dec2"]["b"].reshape(1, FEAT)

    h_ch = x_ch
    feats = []
    for li, layer in enumerate(params["gin"]):
        Cin = h_ch.shape[0]
        agg = _sc_agg(Cin)(h_ch.reshape(Cin * NP, 128),
                           src_rows4[:Cin * ER], dst_rows, zeros_np)
        agg = agg.reshape(Cin, NP, 128)
        h2, stats = _mlp_stats(h_ch, agg,
                               layer["lin1"]["W"].astype(jnp.bfloat16),
                               layer["lin1"]["b"].reshape(1, H),
                               layer["lin2"]["W"].astype(jnp.bfloat16),
                               layer["lin2"]["b"].reshape(1, H))
        if li < 2:
            h_ch = _norm_chunk(h2, stats, layer["gamma"].reshape(1, H),
                               layer["beta"].reshape(1, H))
            feats.append(h_ch)
        else:
            h23, st3 = h2, stats
            g3 = layer["gamma"].reshape(1, H)
            b3 = layer["beta"].reshape(1, H)

    zn, nkl, si, sm = _heads(feats[0], feats[1], h23, st3, g3, b3, wh, bh,
                             eps_n, batch_b)
    loss = _decoder(zn, batch_b, x_pad, si, sm, eps_g, nkl,
                    w1d, b1d, w2d, b2d)
    return loss[0, 0]
